# baseline (speedup 1.0000x reference)
import functools

import jax
import jax.numpy as jnp
from jax import lax
from jax.experimental import pallas as pl
from jax.experimental.pallas import tpu as pltpu
from jax.experimental.pallas import tpu_sc as plsc

EMBED = 64
LANES = 16
NC, NS = 2, 16
NW = NC * NS
BATCH = 16
CTX = 2048
TOTAL = BATCH * CTX
BPW = TOTAL // NW
CHUNK = 128
NCHUNK = BPW // CHUNK
SCALE = 8.0


def _sc_embed(X, tab, pos):
    mesh = plsc.VectorSubcoreMesh(core_axis_name="c", subcore_axis_name="s")

    @functools.partial(
        pl.kernel,
        out_type=jax.ShapeDtypeStruct((BATCH, CTX, EMBED), jnp.float32),
        mesh=mesh,
        scratch_types=[
            pltpu.VMEM((BPW,), jnp.int32),
            pltpu.VMEM((CHUNK, EMBED), jnp.float32),
            pltpu.VMEM((CHUNK, EMBED), jnp.float32),
            pltpu.VMEM((CHUNK, EMBED), jnp.float32),
            pltpu.VMEM((CHUNK, EMBED), jnp.float32),
            pltpu.SemaphoreType.DMA,
            pltpu.SemaphoreType.DMA,
            pltpu.SemaphoreType.DMA,
            pltpu.SemaphoreType.DMA,
            pltpu.SemaphoreType.DMA,
            pltpu.SemaphoreType.DMA,
        ],
        compiler_params=pltpu.CompilerParams(
            needs_layout_passes=False, use_tc_tiling_on_sc=True),
    )
    def k(x_hbm, tab_hbm, pos_hbm, out_hbm, idx_v, buf_a, buf_b,
          pos_a, pos_b, gsem_a, gsem_b, wsem_a, wsem_b, psem_a, psem_b):
        wid = lax.axis_index("s") * NC + lax.axis_index("c")
        b = wid // 2
        t0 = (wid % 2) * BPW

        bufs = (buf_a, buf_b)
        poss = (pos_a, pos_b)
        gsems = (gsem_a, gsem_b)
        wsems = (wsem_a, wsem_b)
        psems = (psem_a, psem_b)

        pltpu.sync_copy(x_hbm.at[b, pl.ds(t0, BPW)], idx_v)

        def fire_chunk(j, par):
            buf, gsem = bufs[par], gsems[par]
            pltpu.async_copy(
                pos_hbm.at[0, pl.ds(t0 + j * CHUNK, CHUNK)],
                poss[par], psems[par])

            def fire_group(i, _):
                v16 = idx_v[pl.ds(j * CHUNK + i * LANES, LANES)]
                for r in range(LANES):
                    pltpu.async_copy(
                        tab_hbm.at[pl.ds(v16[r], 1)],
                        buf.at[pl.ds(i * LANES + r, 1)], gsem)
                return 0

            lax.fori_loop(0, CHUNK // LANES, fire_group, 0)

        def drain_chunk(par):
            pltpu.make_async_copy(
                tab_hbm.at[pl.ds(0, CHUNK)], bufs[par], gsems[par]).wait()
            pltpu.make_async_copy(
                pos_hbm.at[0, pl.ds(0, CHUNK)], poss[par], psems[par]).wait()

        fire_chunk(0, 0)

        wb = [None] * NCHUNK
        for j in range(NCHUNK):
            par = j % 2
            if j + 1 < NCHUNK:
                if j >= 1 and wb[j - 1] is not None:
                    wb[j - 1].wait()
                    wb[j - 1] = None
                fire_chunk(j + 1, 1 - par)
            drain_chunk(par)

            buf, posb = bufs[par], poss[par]

            def row_body(r, _):
                for c in range(EMBED // LANES):
                    sl = (r, pl.ds(c * LANES, LANES))
                    buf[sl] = buf[sl] * SCALE + posb[sl]
                return 0

            lax.fori_loop(0, CHUNK, row_body, 0)
            wb[j] = pltpu.async_copy(
                buf, out_hbm.at[b, pl.ds(t0 + j * CHUNK, CHUNK)], wsems[par])

        for h in wb:
            if h is not None:
                h.wait()

    return k(X, tab, pos)


def kernel(X, tok_table, pos_embedding):
    return _sc_embed(X, tok_table * 1.0, pos_embedding)
